# baseline (speedup 1.0000x reference)
import functools

import jax
import jax.numpy as jnp
from jax import lax
from jax.experimental import pallas as pl
from jax.experimental.pallas import tpu as pltpu
from jax.experimental.pallas import tpu_sc as plsc

N = 10000
D = 128
NC, NS = 2, 16
NW = NC * NS

NPAD = 10240
RPS = NPAD // NS

E = 320000
EPR = 80
IBR = 16
EPROWS = NW * EPR
EPAD = EPROWS * 128
EPR0, EPR1 = 32, 128

L = 200000
LPR = 56
LPROWS = NW * LPR
LPAD = LPROWS * 128
LBR = 16
LPR0, LPR1 = 16, 96

BM = 1000
DM = 2048
_MM = functools.partial(jnp.dot, precision=lax.Precision.HIGHEST,
                        preferred_element_type=jnp.float32)


def _agg_body(table, src2d, dst2d, zeros_hbm, out,
              src_v, dst_v, rows0, rows1, accum, gs0, gs1, ss0, ss1):
    c = lax.axis_index("c")
    s = lax.axis_index("s")
    wid = c * NS + s
    rows = (rows0, rows1)
    gsem = (gs0, gs1)
    ssem = (ss0, ss1)
    pltpu.sync_copy(zeros_hbm, accum.at[pl.ds(s * RPS, RPS)])
    plsc.subcore_barrier()
    base = jnp.where(c == 0, s * EPR0, NS * EPR0 + s * EPR1)

    def g_issue(v, sl):
        pltpu.async_copy(table.at[src_v.at[v]], rows[sl], gsem[sl])

    def g_wait(v, sl):
        pltpu.make_async_copy(table.at[src_v.at[v]], rows[sl],
                              gsem[sl]).wait()

    def s_issue(v, sl):
        pltpu.async_copy(rows[sl], accum.at[dst_v.at[v]], ssem[sl],
                         add=True)

    def s_wait(v, sl):
        pltpu.make_async_copy(rows[sl], accum.at[dst_v.at[v]],
                              ssem[sl]).wait()

    def block(blk, carry):
        off = base + blk * IBR
        pltpu.sync_copy(src2d.at[pl.ds(off, IBR)], src_v)
        pltpu.sync_copy(dst2d.at[pl.ds(off, IBR)], dst_v)
        g_issue(0, 0)
        g_issue(1, 1)
        g_wait(0, 0)
        s_issue(0, 0)

        def chunk(i, carry):
            for b in range(2):
                v = 1 + 2 * i + b
                sl = (1 + b) % 2
                nsl = b
                s_wait(v - 1, nsl)
                g_issue(v + 1, nsl)
                g_wait(v, sl)
                s_issue(v, sl)
            return carry

        lax.fori_loop(0, (IBR - 2) // 2, chunk, 0)
        g_wait(IBR - 1, (IBR - 1) % 2)
        s_issue(IBR - 1, (IBR - 1) % 2)
        s_wait(IBR - 2, (IBR - 2) % 2)
        s_wait(IBR - 1, (IBR - 1) % 2)
        return carry

    nblk = jnp.where(c == 0, EPR0 // IBR, EPR1 // IBR)
    lax.fori_loop(0, nblk, block, 0)
    plsc.subcore_barrier()
    pltpu.sync_copy(accum.at[pl.ds(s * RPS, RPS)],
                    out.at[c, pl.ds(s * RPS, RPS)])


def _make_agg():
    mesh = plsc.VectorSubcoreMesh(core_axis_name="c", subcore_axis_name="s")
    return pl.kernel(
        _agg_body,
        out_type=jax.ShapeDtypeStruct((NC, NPAD, D), jnp.float32),
        mesh=mesh,
        scratch_types=[
            pltpu.VMEM((IBR, 128), jnp.int32),
            pltpu.VMEM((IBR, 128), jnp.int32),
            pltpu.VMEM((128, D), jnp.float32),
            pltpu.VMEM((128, D), jnp.float32),
            pltpu.VMEM_SHARED((NPAD, D), jnp.float32),
            pltpu.SemaphoreType.DMA,
            pltpu.SemaphoreType.DMA,
            pltpu.SemaphoreType.DMA,
            pltpu.SemaphoreType.DMA,
        ],
    )


def _cnt_body(dst_m, dst_r, zeros_hbm, ones_hbm, out,
              dst_v, ones_v, accum, csem):
    c = lax.axis_index("c")
    s = lax.axis_index("s")
    wid = c * NS + s
    base = wid * EPR
    pltpu.sync_copy(ones_hbm, ones_v)
    for k, dst2d in enumerate((dst_m, dst_r)):
        pltpu.sync_copy(zeros_hbm, accum.at[pl.ds(s * RPS, RPS)])
        pltpu.sync_copy(dst2d.at[pl.ds(base, EPR)], dst_v)
        plsc.subcore_barrier()

        pltpu.async_copy(ones_v, accum.at[dst_v.at[0]], csem, add=True)

        def body(v, carry):
            @pl.when(v + 1 < EPR)
            def _():
                pltpu.async_copy(ones_v, accum.at[dst_v.at[v + 1]], csem,
                                 add=True)

            pltpu.make_async_copy(ones_v, accum.at[dst_v.at[v]],
                                  csem).wait()
            return carry

        lax.fori_loop(0, EPR, body, 0)
        plsc.subcore_barrier()
        pltpu.sync_copy(accum.at[pl.ds(s * RPS, RPS)],
                        out.at[k, c, pl.ds(s * RPS, RPS)])
        plsc.subcore_barrier()


def _make_cnt():
    mesh = plsc.VectorSubcoreMesh(core_axis_name="c", subcore_axis_name="s")
    return pl.kernel(
        _cnt_body,
        out_type=jax.ShapeDtypeStruct((2, NC, NPAD, D), jnp.float32),
        mesh=mesh,
        scratch_types=[
            pltpu.VMEM((EPR, 128), jnp.int32),
            pltpu.VMEM((128, D), jnp.float32),
            pltpu.VMEM_SHARED((NPAD, D), jnp.float32),
            pltpu.SemaphoreType.DMA,
        ],
    )


def _dec_body(uod, udt, row2d, col2d, ga, gb, ridx, cidx,
              a0, a1, b0, b1, ga0, ga1, gb0, gb1, wa0, wa1, wb0, wb1):
    c = lax.axis_index("c")
    s = lax.axis_index("s")
    base = jnp.where(c == 0, s * LPR0, NS * LPR0 + s * LPR1)
    A = (a0, a1)
    B = (b0, b1)
    GA = (ga0, ga1)
    GB = (gb0, gb1)
    WA = (wa0, wa1)
    WB = (wb0, wb1)

    def block(blk, carry):
        off = base + blk * LBR
        pltpu.sync_copy(row2d.at[pl.ds(off, LBR)], ridx)
        pltpu.sync_copy(col2d.at[pl.ds(off, LBR)], cidx)

        def g_issue(v, sl):
            pltpu.async_copy(uod.at[ridx.at[v]], A[sl], GA[sl])
            pltpu.async_copy(udt.at[cidx.at[v]], B[sl], GB[sl])

        def g_wait(v, sl):
            pltpu.make_async_copy(uod.at[ridx.at[v]], A[sl], GA[sl]).wait()
            pltpu.make_async_copy(udt.at[cidx.at[v]], B[sl], GB[sl]).wait()

        def w_issue(v, sl):
            pltpu.async_copy(A[sl], ga.at[pl.ds((off + v) * 128, 128)],
                             WA[sl])
            pltpu.async_copy(B[sl], gb.at[pl.ds((off + v) * 128, 128)],
                             WB[sl])

        def w_wait(v, sl):
            pltpu.make_async_copy(A[sl],
                                  ga.at[pl.ds((off + v) * 128, 128)],
                                  WA[sl]).wait()
            pltpu.make_async_copy(B[sl],
                                  gb.at[pl.ds((off + v) * 128, 128)],
                                  WB[sl]).wait()

        g_issue(0, 0)
        g_issue(1, 1)
        g_wait(0, 0)
        w_issue(0, 0)

        def chunk(i, carry2):
            for b in range(2):
                v = 1 + 2 * i + b
                sl = (1 + b) % 2
                nsl = b
                w_wait(v - 1, nsl)
                g_issue(v + 1, nsl)
                g_wait(v, sl)
                w_issue(v, sl)
            return carry2

        lax.fori_loop(0, (LBR - 2) // 2, chunk, 0)
        g_wait(LBR - 1, (LBR - 1) % 2)
        w_issue(LBR - 1, (LBR - 1) % 2)
        w_wait(LBR - 2, (LBR - 2) % 2)
        w_wait(LBR - 1, (LBR - 1) % 2)
        return carry

    nblk = jnp.where(c == 0, LPR0 // LBR, LPR1 // LBR)
    lax.fori_loop(0, nblk, block, 0)


def _make_dec():
    mesh = plsc.VectorSubcoreMesh(core_axis_name="c", subcore_axis_name="s")
    return pl.kernel(
        _dec_body,
        out_type=(jax.ShapeDtypeStruct((LPAD, D), jnp.float32),
                  jax.ShapeDtypeStruct((LPAD, D), jnp.float32)),
        mesh=mesh,
        scratch_types=[
            pltpu.VMEM((LBR, 128), jnp.int32),
            pltpu.VMEM((LBR, 128), jnp.int32),
            pltpu.VMEM((128, D), jnp.float32),
            pltpu.VMEM((128, D), jnp.float32),
            pltpu.VMEM((128, D), jnp.float32),
            pltpu.VMEM((128, D), jnp.float32),
        ] + [pltpu.SemaphoreType.DMA] * 8,
    )


def _t1_body(pa0, pa1, cm0, cm1, x, w1l_od, b1_od, w1r_od,
             w1l_dt, b1_dt, w1r_dt, h_out, d1_out):
    cnt = jnp.maximum(cm0[:, :1] + cm1[:, :1], 1.0)
    agg = (pa0[...] + pa1[...]) / cnt
    xb = x[...]
    h_out[...] = jnp.maximum(_MM(agg, w1l_od[...]) + b1_od[...]
                             + _MM(xb, w1r_od[...]), 0.0)
    d1_out[...] = jnp.maximum(_MM(agg, w1l_dt[...]) + b1_dt[...]
                              + _MM(xb, w1r_dt[...]), 0.0)


def _t1(pa0, pa1, cm0, cm1, x, w1l_od, b1_od, w1r_od, w1l_dt, b1_dt, w1r_dt):
    grid = N // BM
    blk_d = pl.BlockSpec((BM, D), lambda i: (i, 0))
    blk_w = pl.BlockSpec((D, D), lambda i: (0, 0))
    blk_b = pl.BlockSpec((1, D), lambda i: (0, 0))
    return pl.pallas_call(
        _t1_body,
        grid=(grid,),
        in_specs=[blk_d, blk_d, blk_d, blk_d, blk_d,
                  blk_w, blk_b, blk_w, blk_w, blk_b, blk_w],
        out_specs=[blk_d, blk_d],
        out_shape=[jax.ShapeDtypeStruct((N, D), jnp.float32),
                   jax.ShapeDtypeStruct((N, D), jnp.float32)],
    )(pa0, pa1, cm0, cm1, x, w1l_od, b1_od, w1r_od, w1l_dt, b1_dt, w1r_dt)


def _t2_body(pb0, pb1, cr0, cr1, pc0, pc1, cm0, cm1, xod, d1,
             w2l, b2, w2r, w3l, b3, w3r, wlin_od, blin_od,
             dw2l, db2, dw2r, wlin_dt, blin_dt,
             w1a, w1b, db1, uod_out, udt_out):
    cntb = jnp.maximum(cr0[:, :1] + cr1[:, :1], 1.0)
    aggh = (pb0[...] + pb1[...]) / cntb
    cntc = jnp.maximum(cm0[:, :1] + cm1[:, :1], 1.0)
    aggd = (pc0[...] + pc1[...]) / cntc
    xb = xod[...]
    d1b = d1[...]
    od1 = jnp.maximum(_MM(aggh, w2l[...]) + b2[...] + _MM(xb, w2r[...]), 0.0)
    od2 = jnp.maximum(_MM(aggh, w3l[...]) + b3[...] + _MM(od1, w3r[...]), 0.0)
    d2 = jnp.maximum(_MM(aggd, dw2l[...]) + db2[...] + _MM(d1b, dw2r[...]),
                     0.0)
    z_od = _MM(od2, wlin_od[...]) + blin_od[...]
    z_dt = _MM(d2, wlin_dt[...]) + blin_dt[...]
    uod_out[...] = _MM(z_od, w1a[...])
    udt_out[...] = _MM(z_dt, w1b[...]) + db1[...]


def _t2(*args):
    grid = N // BM
    blk_d = pl.BlockSpec((BM, D), lambda i: (i, 0))
    blk_w = pl.BlockSpec((D, D), lambda i: (0, 0))
    blk_b = pl.BlockSpec((1, D), lambda i: (0, 0))
    wspecs = [blk_w, blk_b, blk_w, blk_w, blk_b, blk_w, blk_w, blk_b,
              blk_w, blk_b, blk_w, blk_w, blk_b, blk_w, blk_w, blk_b]
    return pl.pallas_call(
        _t2_body,
        grid=(grid,),
        in_specs=[blk_d] * 10 + wspecs,
        out_specs=[blk_d, blk_d],
        out_shape=[jax.ShapeDtypeStruct((N, D), jnp.float32),
                   jax.ShapeDtypeStruct((N, D), jnp.float32)],
    )(*args)


def _t3_body(ga, gb, w2row, b2s, out):
    t = jnp.maximum(ga[...] + gb[...], 0.0)
    s = jnp.sum(t * w2row[...], axis=1, keepdims=True) + b2s[...]
    out[...] = jax.nn.sigmoid(s)


def _t3(ga, gb, w2row, b2s):
    grid = LPAD // DM
    blk = pl.BlockSpec((DM, D), lambda i: (i, 0))
    return pl.pallas_call(
        _t3_body,
        grid=(grid,),
        in_specs=[blk, blk, pl.BlockSpec((1, D), lambda i: (0, 0)),
                  pl.BlockSpec((1, 1), lambda i: (0, 0))],
        out_specs=pl.BlockSpec((DM, 1), lambda i: (i, 0)),
        out_shape=jax.ShapeDtypeStruct((LPAD, 1), jnp.float32),
    )(ga, gb, w2row, b2s)


def _pad_edges(ei, pad_len, dummy_dst):
    src = jnp.concatenate([ei[0], jnp.zeros((pad_len,), jnp.int32)])
    dst = jnp.concatenate([ei[1], jnp.full((pad_len,), dummy_dst, jnp.int32)])
    return src.reshape(-1, 128), dst.reshape(-1, 128)


def kernel(x_detector_time, x_od, od_W1l, od_b1, od_W1r, od_W2l, od_b2,
           od_W2r, od_W3l, od_b3, od_W3r, od_Wlin, od_blin, dt_W1l, dt_b1,
           dt_W1r, dt_W2l, dt_b2, dt_W2r, dt_Wlin, dt_blin, dec_W1, dec_b1,
           dec_W2, dec_b2, edge_index_metapath, edge_index_rev_assignment,
           edge_label_index):
    f32 = jnp.float32
    zeros_hbm = jnp.zeros((RPS, D), f32)
    ones_hbm = jnp.ones((128, D), f32)

    src_m, dst_m = _pad_edges(edge_index_metapath, EPAD - E, N)
    src_r, dst_r = _pad_edges(edge_index_rev_assignment, EPAD - E, N)
    row_l, col_l = _pad_edges(edge_label_index, LPAD - L, 0)

    agg = _make_agg()
    cnt = _make_cnt()
    dec = _make_dec()

    row2 = lambda b: b.reshape(1, -1)

    cnts = cnt(dst_m, dst_r, zeros_hbm, ones_hbm)
    pa = agg(x_detector_time, src_m, dst_m, zeros_hbm)
    h, d1 = _t1(pa[0], pa[1], cnts[0, 0], cnts[0, 1], x_detector_time,
                od_W1l, row2(od_b1), od_W1r,
                dt_W1l, row2(dt_b1), dt_W1r)

    pb = agg(h, src_r, dst_r, zeros_hbm)
    pc = agg(d1, src_m, dst_m, zeros_hbm)

    u_od, u_dt = _t2(pb[0], pb[1], cnts[1, 0], cnts[1, 1],
                     pc[0], pc[1], cnts[0, 0], cnts[0, 1], x_od, d1,
                     od_W2l, row2(od_b2), od_W2r,
                     od_W3l, row2(od_b3), od_W3r,
                     od_Wlin, row2(od_blin),
                     dt_W2l, row2(dt_b2), dt_W2r,
                     dt_Wlin, row2(dt_blin),
                     dec_W1[:D], dec_W1[D:], row2(dec_b1))

    ga, gb = dec(u_od, u_dt, row_l, col_l)
    out = _t3(ga, gb, dec_W2.reshape(1, D), dec_b2.reshape(1, 1))
    return out.reshape(-1)[:L]


# speedup vs baseline: 1.1006x; 1.1006x over previous
import functools

import jax
import jax.numpy as jnp
from jax import lax
from jax.experimental import pallas as pl
from jax.experimental.pallas import tpu as pltpu
from jax.experimental.pallas import tpu_sc as plsc

N = 10000
D = 128
NC, NS = 2, 16
NW = NC * NS

NPAD = 10240
RPS = NPAD // NS

E = 320000
EPR = 80
IBR = 16
EPROWS = NW * EPR
EPAD = EPROWS * 128
EPR0, EPR1 = 80, 80

L = 200000
LPR = 56
LPROWS = NW * LPR
LPAD = LPROWS * 128
LBR = 8
LPR0, LPR1 = 56, 56

BM = 1000
DM = 2048
_MM = functools.partial(jnp.dot, precision=lax.Precision.HIGHEST,
                        preferred_element_type=jnp.float32)


def _agg_body(table, src2d, dst2d, zeros_hbm, out,
              src_v, dst_v, rows0, rows1, accum, gs0, gs1, ss0, ss1):
    c = lax.axis_index("c")
    s = lax.axis_index("s")
    wid = c * NS + s
    rows = (rows0, rows1)
    gsem = (gs0, gs1)
    ssem = (ss0, ss1)
    pltpu.sync_copy(zeros_hbm, accum.at[pl.ds(s * RPS, RPS)])
    plsc.subcore_barrier()
    base = jnp.where(c == 0, s * EPR0, NS * EPR0 + s * EPR1)

    def g_issue(v, sl):
        pltpu.async_copy(table.at[src_v.at[v]], rows[sl], gsem[sl])

    def g_wait(v, sl):
        pltpu.make_async_copy(table.at[src_v.at[v]], rows[sl],
                              gsem[sl]).wait()

    def s_issue(v, sl):
        pltpu.async_copy(rows[sl], accum.at[dst_v.at[v]], ssem[sl],
                         add=True)

    def s_wait(v, sl):
        pltpu.make_async_copy(rows[sl], accum.at[dst_v.at[v]],
                              ssem[sl]).wait()

    def block(blk, carry):
        off = base + blk * IBR
        pltpu.sync_copy(src2d.at[pl.ds(off, IBR)], src_v)
        pltpu.sync_copy(dst2d.at[pl.ds(off, IBR)], dst_v)
        g_issue(0, 0)
        g_issue(1, 1)
        g_wait(0, 0)
        s_issue(0, 0)

        def chunk(i, carry):
            for b in range(2):
                v = 1 + 2 * i + b
                sl = (1 + b) % 2
                nsl = b
                s_wait(v - 1, nsl)
                g_issue(v + 1, nsl)
                g_wait(v, sl)
                s_issue(v, sl)
            return carry

        lax.fori_loop(0, (IBR - 2) // 2, chunk, 0)
        g_wait(IBR - 1, (IBR - 1) % 2)
        s_issue(IBR - 1, (IBR - 1) % 2)
        s_wait(IBR - 2, (IBR - 2) % 2)
        s_wait(IBR - 1, (IBR - 1) % 2)
        return carry

    nblk = jnp.where(c == 0, EPR0 // IBR, EPR1 // IBR)
    lax.fori_loop(0, nblk, block, 0)
    plsc.subcore_barrier()
    pltpu.sync_copy(accum.at[pl.ds(s * RPS, RPS)],
                    out.at[c, pl.ds(s * RPS, RPS)])


def _make_agg():
    mesh = plsc.VectorSubcoreMesh(core_axis_name="c", subcore_axis_name="s")
    return pl.kernel(
        _agg_body,
        out_type=jax.ShapeDtypeStruct((NC, NPAD, D), jnp.float32),
        mesh=mesh,
        scratch_types=[
            pltpu.VMEM((IBR, 128), jnp.int32),
            pltpu.VMEM((IBR, 128), jnp.int32),
            pltpu.VMEM((128, D), jnp.float32),
            pltpu.VMEM((128, D), jnp.float32),
            pltpu.VMEM_SHARED((NPAD, D), jnp.float32),
            pltpu.SemaphoreType.DMA,
            pltpu.SemaphoreType.DMA,
            pltpu.SemaphoreType.DMA,
            pltpu.SemaphoreType.DMA,
        ],
    )


def _cnt_body(dst_m, dst_r, zeros_hbm, ones_hbm, out,
              dst_v, ones_v, accum, csem):
    c = lax.axis_index("c")
    s = lax.axis_index("s")
    wid = c * NS + s
    base = wid * EPR
    pltpu.sync_copy(ones_hbm, ones_v)
    for k, dst2d in enumerate((dst_m, dst_r)):
        pltpu.sync_copy(zeros_hbm, accum.at[pl.ds(s * RPS, RPS)])
        pltpu.sync_copy(dst2d.at[pl.ds(base, EPR)], dst_v)
        plsc.subcore_barrier()

        pltpu.async_copy(ones_v, accum.at[dst_v.at[0]], csem, add=True)

        def body(v, carry):
            @pl.when(v + 1 < EPR)
            def _():
                pltpu.async_copy(ones_v, accum.at[dst_v.at[v + 1]], csem,
                                 add=True)

            pltpu.make_async_copy(ones_v, accum.at[dst_v.at[v]],
                                  csem).wait()
            return carry

        lax.fori_loop(0, EPR, body, 0)
        plsc.subcore_barrier()
        pltpu.sync_copy(accum.at[pl.ds(s * RPS, RPS)],
                        out.at[k, c, pl.ds(s * RPS, RPS)])
        plsc.subcore_barrier()


def _make_cnt():
    mesh = plsc.VectorSubcoreMesh(core_axis_name="c", subcore_axis_name="s")
    return pl.kernel(
        _cnt_body,
        out_type=jax.ShapeDtypeStruct((2, NC, NPAD, D), jnp.float32),
        mesh=mesh,
        scratch_types=[
            pltpu.VMEM((EPR, 128), jnp.int32),
            pltpu.VMEM((128, D), jnp.float32),
            pltpu.VMEM_SHARED((NPAD, D), jnp.float32),
            pltpu.SemaphoreType.DMA,
        ],
    )


def _dec_body(uod, udt, row2d, col2d, ga, gb, ridx, cidx,
              a0, a1, b0, b1, ga0, ga1, gb0, gb1, wa0, wa1, wb0, wb1):
    c = lax.axis_index("c")
    s = lax.axis_index("s")
    base = jnp.where(c == 0, s * LPR0, NS * LPR0 + s * LPR1)
    A = (a0, a1)
    B = (b0, b1)
    GA = (ga0, ga1)
    GB = (gb0, gb1)
    WA = (wa0, wa1)
    WB = (wb0, wb1)

    def block(blk, carry):
        off = base + blk * LBR
        pltpu.sync_copy(row2d.at[pl.ds(off, LBR)], ridx)
        pltpu.sync_copy(col2d.at[pl.ds(off, LBR)], cidx)

        def g_issue(v, sl):
            pltpu.async_copy(uod.at[ridx.at[v]], A[sl], GA[sl])
            pltpu.async_copy(udt.at[cidx.at[v]], B[sl], GB[sl])

        def g_wait(v, sl):
            pltpu.make_async_copy(uod.at[ridx.at[v]], A[sl], GA[sl]).wait()
            pltpu.make_async_copy(udt.at[cidx.at[v]], B[sl], GB[sl]).wait()

        def w_issue(v, sl):
            pltpu.async_copy(A[sl], ga.at[pl.ds((off + v) * 128, 128)],
                             WA[sl])
            pltpu.async_copy(B[sl], gb.at[pl.ds((off + v) * 128, 128)],
                             WB[sl])

        def w_wait(v, sl):
            pltpu.make_async_copy(A[sl],
                                  ga.at[pl.ds((off + v) * 128, 128)],
                                  WA[sl]).wait()
            pltpu.make_async_copy(B[sl],
                                  gb.at[pl.ds((off + v) * 128, 128)],
                                  WB[sl]).wait()

        g_issue(0, 0)
        g_issue(1, 1)
        g_wait(0, 0)
        w_issue(0, 0)

        def chunk(i, carry2):
            for b in range(2):
                v = 1 + 2 * i + b
                sl = (1 + b) % 2
                nsl = b
                w_wait(v - 1, nsl)
                g_issue(v + 1, nsl)
                g_wait(v, sl)
                w_issue(v, sl)
            return carry2

        lax.fori_loop(0, (LBR - 2) // 2, chunk, 0)
        g_wait(LBR - 1, (LBR - 1) % 2)
        w_issue(LBR - 1, (LBR - 1) % 2)
        w_wait(LBR - 2, (LBR - 2) % 2)
        w_wait(LBR - 1, (LBR - 1) % 2)
        return carry

    nblk = jnp.where(c == 0, LPR0 // LBR, LPR1 // LBR)
    lax.fori_loop(0, nblk, block, 0)


def _make_dec():
    mesh = plsc.VectorSubcoreMesh(core_axis_name="c", subcore_axis_name="s")
    return pl.kernel(
        _dec_body,
        out_type=(jax.ShapeDtypeStruct((LPAD, D), jnp.float32),
                  jax.ShapeDtypeStruct((LPAD, D), jnp.float32)),
        mesh=mesh,
        scratch_types=[
            pltpu.VMEM((LBR, 128), jnp.int32),
            pltpu.VMEM((LBR, 128), jnp.int32),
            pltpu.VMEM((128, D), jnp.float32),
            pltpu.VMEM((128, D), jnp.float32),
            pltpu.VMEM((128, D), jnp.float32),
            pltpu.VMEM((128, D), jnp.float32),
        ] + [pltpu.SemaphoreType.DMA] * 8,
    )


def _t1_body(pa0, pa1, cm0, cm1, x, w1l_od, b1_od, w1r_od,
             w1l_dt, b1_dt, w1r_dt, h_out, d1_out):
    cnt = jnp.maximum(cm0[:, :1] + cm1[:, :1], 1.0)
    agg = (pa0[...] + pa1[...]) / cnt
    xb = x[...]
    h_out[...] = jnp.maximum(_MM(agg, w1l_od[...]) + b1_od[...]
                             + _MM(xb, w1r_od[...]), 0.0)
    d1_out[...] = jnp.maximum(_MM(agg, w1l_dt[...]) + b1_dt[...]
                              + _MM(xb, w1r_dt[...]), 0.0)


def _t1(pa0, pa1, cm0, cm1, x, w1l_od, b1_od, w1r_od, w1l_dt, b1_dt, w1r_dt):
    grid = N // BM
    blk_d = pl.BlockSpec((BM, D), lambda i: (i, 0))
    blk_w = pl.BlockSpec((D, D), lambda i: (0, 0))
    blk_b = pl.BlockSpec((1, D), lambda i: (0, 0))
    return pl.pallas_call(
        _t1_body,
        grid=(grid,),
        in_specs=[blk_d, blk_d, blk_d, blk_d, blk_d,
                  blk_w, blk_b, blk_w, blk_w, blk_b, blk_w],
        out_specs=[blk_d, blk_d],
        out_shape=[jax.ShapeDtypeStruct((N, D), jnp.float32),
                   jax.ShapeDtypeStruct((N, D), jnp.float32)],
    )(pa0, pa1, cm0, cm1, x, w1l_od, b1_od, w1r_od, w1l_dt, b1_dt, w1r_dt)


def _t2_body(pb0, pb1, cr0, cr1, pc0, pc1, cm0, cm1, xod, d1,
             w2l, b2, w2r, w3l, b3, w3r, wlin_od, blin_od,
             dw2l, db2, dw2r, wlin_dt, blin_dt,
             w1a, w1b, db1, uod_out, udt_out):
    cntb = jnp.maximum(cr0[:, :1] + cr1[:, :1], 1.0)
    aggh = (pb0[...] + pb1[...]) / cntb
    cntc = jnp.maximum(cm0[:, :1] + cm1[:, :1], 1.0)
    aggd = (pc0[...] + pc1[...]) / cntc
    xb = xod[...]
    d1b = d1[...]
    od1 = jnp.maximum(_MM(aggh, w2l[...]) + b2[...] + _MM(xb, w2r[...]), 0.0)
    od2 = jnp.maximum(_MM(aggh, w3l[...]) + b3[...] + _MM(od1, w3r[...]), 0.0)
    d2 = jnp.maximum(_MM(aggd, dw2l[...]) + db2[...] + _MM(d1b, dw2r[...]),
                     0.0)
    z_od = _MM(od2, wlin_od[...]) + blin_od[...]
    z_dt = _MM(d2, wlin_dt[...]) + blin_dt[...]
    uod_out[...] = _MM(z_od, w1a[...])
    udt_out[...] = _MM(z_dt, w1b[...]) + db1[...]


def _t2(*args):
    grid = N // BM
    blk_d = pl.BlockSpec((BM, D), lambda i: (i, 0))
    blk_w = pl.BlockSpec((D, D), lambda i: (0, 0))
    blk_b = pl.BlockSpec((1, D), lambda i: (0, 0))
    wspecs = [blk_w, blk_b, blk_w, blk_w, blk_b, blk_w, blk_w, blk_b,
              blk_w, blk_b, blk_w, blk_w, blk_b, blk_w, blk_w, blk_b]
    return pl.pallas_call(
        _t2_body,
        grid=(grid,),
        in_specs=[blk_d] * 10 + wspecs,
        out_specs=[blk_d, blk_d],
        out_shape=[jax.ShapeDtypeStruct((N, D), jnp.float32),
                   jax.ShapeDtypeStruct((N, D), jnp.float32)],
    )(*args)


def _t3_body(ga, gb, w2row, b2s, out):
    t = jnp.maximum(ga[...] + gb[...], 0.0)
    s = jnp.sum(t * w2row[...], axis=1, keepdims=True) + b2s[...]
    out[...] = jax.nn.sigmoid(s)


def _t3(ga, gb, w2row, b2s):
    grid = LPAD // DM
    blk = pl.BlockSpec((DM, D), lambda i: (i, 0))
    return pl.pallas_call(
        _t3_body,
        grid=(grid,),
        in_specs=[blk, blk, pl.BlockSpec((1, D), lambda i: (0, 0)),
                  pl.BlockSpec((1, 1), lambda i: (0, 0))],
        out_specs=pl.BlockSpec((DM, 1), lambda i: (i, 0)),
        out_shape=jax.ShapeDtypeStruct((LPAD, 1), jnp.float32),
    )(ga, gb, w2row, b2s)


def _pad_edges(ei, pad_len, dummy_dst):
    src = jnp.concatenate([ei[0], jnp.zeros((pad_len,), jnp.int32)])
    dst = jnp.concatenate([ei[1], jnp.full((pad_len,), dummy_dst, jnp.int32)])
    return src.reshape(-1, 128), dst.reshape(-1, 128)


def kernel(x_detector_time, x_od, od_W1l, od_b1, od_W1r, od_W2l, od_b2,
           od_W2r, od_W3l, od_b3, od_W3r, od_Wlin, od_blin, dt_W1l, dt_b1,
           dt_W1r, dt_W2l, dt_b2, dt_W2r, dt_Wlin, dt_blin, dec_W1, dec_b1,
           dec_W2, dec_b2, edge_index_metapath, edge_index_rev_assignment,
           edge_label_index):
    f32 = jnp.float32
    zeros_hbm = jnp.zeros((RPS, D), f32)
    ones_hbm = jnp.ones((128, D), f32)

    src_m, dst_m = _pad_edges(edge_index_metapath, EPAD - E, N)
    src_r, dst_r = _pad_edges(edge_index_rev_assignment, EPAD - E, N)
    row_l, col_l = _pad_edges(edge_label_index, LPAD - L, 0)

    agg = _make_agg()
    cnt = _make_cnt()
    dec = _make_dec()

    row2 = lambda b: b.reshape(1, -1)

    cnts = cnt(dst_m, dst_r, zeros_hbm, ones_hbm)
    pa = agg(x_detector_time, src_m, dst_m, zeros_hbm)
    h, d1 = _t1(pa[0], pa[1], cnts[0, 0], cnts[0, 1], x_detector_time,
                od_W1l, row2(od_b1), od_W1r,
                dt_W1l, row2(dt_b1), dt_W1r)

    pb = agg(h, src_r, dst_r, zeros_hbm)
    pc = agg(d1, src_m, dst_m, zeros_hbm)

    u_od, u_dt = _t2(pb[0], pb[1], cnts[1, 0], cnts[1, 1],
                     pc[0], pc[1], cnts[0, 0], cnts[0, 1], x_od, d1,
                     od_W2l, row2(od_b2), od_W2r,
                     od_W3l, row2(od_b3), od_W3r,
                     od_Wlin, row2(od_blin),
                     dt_W2l, row2(dt_b2), dt_W2r,
                     dt_Wlin, row2(dt_blin),
                     dec_W1[:D], dec_W1[D:], row2(dec_b1))

    ga, gb = dec(u_od, u_dt, row_l, col_l)
    out = _t3(ga, gb, dec_W2.reshape(1, D), dec_b2.reshape(1, 1))
    return out.reshape(-1)[:L]


# speedup vs baseline: 1.7127x; 1.5561x over previous
import functools

import jax
import jax.numpy as jnp
from jax import lax
from jax.experimental import pallas as pl
from jax.experimental.pallas import tpu as pltpu
from jax.experimental.pallas import tpu_sc as plsc

N = 10000
D = 128
NC, NS = 2, 16
NW = NC * NS

NPAD = 10240
RPS = NPAD // NS

E = 320000
EPR = 80
IBR = 16
EPROWS = NW * EPR
EPAD = EPROWS * 128
EPR0, EPR1 = 80, 80

L = 200000
LPR = 56
LPROWS = NW * LPR
LPAD = LPROWS * 128
LBR = 16
LPW = LPROWS // NS

BM = 1000
DM = 2048
_MM = functools.partial(jnp.dot, precision=lax.Precision.HIGHEST,
                        preferred_element_type=jnp.float32)


def _agg_body(table, src2d, dst2d, zeros_hbm, out,
              src_v, dst_v, rows0, rows1, accum, gs0, gs1, ss0, ss1):
    c = lax.axis_index("c")
    s = lax.axis_index("s")
    wid = c * NS + s
    rows = (rows0, rows1)
    gsem = (gs0, gs1)
    ssem = (ss0, ss1)
    pltpu.sync_copy(zeros_hbm, accum.at[pl.ds(s * RPS, RPS)])
    plsc.subcore_barrier()
    base = jnp.where(c == 0, s * EPR0, NS * EPR0 + s * EPR1)

    def g_issue(v, sl):
        pltpu.async_copy(table.at[src_v.at[v]], rows[sl], gsem[sl])

    def g_wait(v, sl):
        pltpu.make_async_copy(table.at[src_v.at[v]], rows[sl],
                              gsem[sl]).wait()

    def s_issue(v, sl):
        pltpu.async_copy(rows[sl], accum.at[dst_v.at[v]], ssem[sl],
                         add=True)

    def s_wait(v, sl):
        pltpu.make_async_copy(rows[sl], accum.at[dst_v.at[v]],
                              ssem[sl]).wait()

    def block(blk, carry):
        off = base + blk * IBR
        pltpu.sync_copy(src2d.at[pl.ds(off, IBR)], src_v)
        pltpu.sync_copy(dst2d.at[pl.ds(off, IBR)], dst_v)
        g_issue(0, 0)
        g_issue(1, 1)
        g_wait(0, 0)
        s_issue(0, 0)

        def chunk(i, carry):
            for b in range(2):
                v = 1 + 2 * i + b
                sl = (1 + b) % 2
                nsl = b
                s_wait(v - 1, nsl)
                g_issue(v + 1, nsl)
                g_wait(v, sl)
                s_issue(v, sl)
            return carry

        lax.fori_loop(0, (IBR - 2) // 2, chunk, 0)
        g_wait(IBR - 1, (IBR - 1) % 2)
        s_issue(IBR - 1, (IBR - 1) % 2)
        s_wait(IBR - 2, (IBR - 2) % 2)
        s_wait(IBR - 1, (IBR - 1) % 2)
        return carry

    nblk = jnp.where(c == 0, EPR0 // IBR, EPR1 // IBR)
    lax.fori_loop(0, nblk, block, 0)
    plsc.subcore_barrier()
    pltpu.sync_copy(accum.at[pl.ds(s * RPS, RPS)],
                    out.at[c, pl.ds(s * RPS, RPS)])


def _make_agg():
    mesh = plsc.VectorSubcoreMesh(core_axis_name="c", subcore_axis_name="s")
    return pl.kernel(
        _agg_body,
        out_type=jax.ShapeDtypeStruct((NC, NPAD, D), jnp.float32),
        mesh=mesh,
        scratch_types=[
            pltpu.VMEM((IBR, 128), jnp.int32),
            pltpu.VMEM((IBR, 128), jnp.int32),
            pltpu.VMEM((128, D), jnp.float32),
            pltpu.VMEM((128, D), jnp.float32),
            pltpu.VMEM_SHARED((NPAD, D), jnp.float32),
            pltpu.SemaphoreType.DMA,
            pltpu.SemaphoreType.DMA,
            pltpu.SemaphoreType.DMA,
            pltpu.SemaphoreType.DMA,
        ],
    )


def _cnt_body(dst_m, dst_r, zeros_hbm, ones_hbm, out,
              dst_v, ones_v, accum, csem):
    c = lax.axis_index("c")
    s = lax.axis_index("s")
    wid = c * NS + s
    base = wid * EPR
    pltpu.sync_copy(ones_hbm, ones_v)
    for k, dst2d in enumerate((dst_m, dst_r)):
        pltpu.sync_copy(zeros_hbm, accum.at[pl.ds(s * RPS, RPS)])
        pltpu.sync_copy(dst2d.at[pl.ds(base, EPR)], dst_v)
        plsc.subcore_barrier()

        pltpu.async_copy(ones_v, accum.at[dst_v.at[0]], csem, add=True)

        def body(v, carry):
            @pl.when(v + 1 < EPR)
            def _():
                pltpu.async_copy(ones_v, accum.at[dst_v.at[v + 1]], csem,
                                 add=True)

            pltpu.make_async_copy(ones_v, accum.at[dst_v.at[v]],
                                  csem).wait()
            return carry

        lax.fori_loop(0, EPR, body, 0)
        plsc.subcore_barrier()
        pltpu.sync_copy(accum.at[pl.ds(s * RPS, RPS)],
                        out.at[k, c, pl.ds(s * RPS, RPS)])
        plsc.subcore_barrier()


def _make_cnt():
    mesh = plsc.VectorSubcoreMesh(core_axis_name="c", subcore_axis_name="s")
    return pl.kernel(
        _cnt_body,
        out_type=jax.ShapeDtypeStruct((2, NC, NPAD, D), jnp.float32),
        mesh=mesh,
        scratch_types=[
            pltpu.VMEM((EPR, 128), jnp.int32),
            pltpu.VMEM((128, D), jnp.float32),
            pltpu.VMEM_SHARED((NPAD, D), jnp.float32),
            pltpu.SemaphoreType.DMA,
        ],
    )


def _dec_body(uod_p, udt_p, row2d, col2d, ga, gb, idxv,
              r0, r1, gs0, gs1, ws0, ws1, table):
    c = lax.axis_index("c")
    s = lax.axis_index("s")
    R = (r0, r1)
    GS = (gs0, gs1)
    WS = (ws0, ws1)

    @pl.when(c == 0)
    def _():
        pltpu.sync_copy(uod_p.at[pl.ds(s * RPS, RPS)],
                        table.at[pl.ds(s * RPS, RPS)])

    @pl.when(c == 1)
    def _():
        pltpu.sync_copy(udt_p.at[pl.ds(s * RPS, RPS)],
                        table.at[pl.ds(s * RPS, RPS)])

    plsc.subcore_barrier()
    base = s * LPW

    def run(idx2d, out):
        def g_issue(v, sl):
            pltpu.async_copy(table.at[idxv.at[v]], R[sl], GS[sl])

        def g_wait(v, sl):
            pltpu.make_async_copy(table.at[idxv.at[v]], R[sl],
                                  GS[sl]).wait()

        def block(blk, carry):
            off = base + blk * LBR
            pltpu.sync_copy(idx2d.at[pl.ds(off, LBR)], idxv)

            def w_issue(v, sl):
                pltpu.async_copy(R[sl], out.at[pl.ds((off + v) * 128, 128)],
                                 WS[sl])

            def w_wait(v, sl):
                pltpu.make_async_copy(R[sl],
                                      out.at[pl.ds((off + v) * 128, 128)],
                                      WS[sl]).wait()

            g_issue(0, 0)
            g_issue(1, 1)
            g_wait(0, 0)
            w_issue(0, 0)

            def chunk(i, carry2):
                for b in range(2):
                    v = 1 + 2 * i + b
                    sl = (1 + b) % 2
                    nsl = b
                    w_wait(v - 1, nsl)
                    g_issue(v + 1, nsl)
                    g_wait(v, sl)
                    w_issue(v, sl)
                return carry2

            lax.fori_loop(0, (LBR - 2) // 2, chunk, 0)
            g_wait(LBR - 1, (LBR - 1) % 2)
            w_issue(LBR - 1, (LBR - 1) % 2)
            w_wait(LBR - 2, (LBR - 2) % 2)
            w_wait(LBR - 1, (LBR - 1) % 2)
            return carry

        lax.fori_loop(0, LPW // LBR, block, 0)

    @pl.when(c == 0)
    def _():
        run(row2d, ga)

    @pl.when(c == 1)
    def _():
        run(col2d, gb)


def _make_dec():
    mesh = plsc.VectorSubcoreMesh(core_axis_name="c", subcore_axis_name="s")
    return pl.kernel(
        _dec_body,
        out_type=(jax.ShapeDtypeStruct((LPAD, D), jnp.float32),
                  jax.ShapeDtypeStruct((LPAD, D), jnp.float32)),
        mesh=mesh,
        scratch_types=[
            pltpu.VMEM((LBR, 128), jnp.int32),
            pltpu.VMEM((128, D), jnp.float32),
            pltpu.VMEM((128, D), jnp.float32),
            pltpu.SemaphoreType.DMA,
            pltpu.SemaphoreType.DMA,
            pltpu.SemaphoreType.DMA,
            pltpu.SemaphoreType.DMA,
            pltpu.VMEM_SHARED((NPAD, D), jnp.float32),
        ],
    )


def _t1_body(pa0, pa1, cm0, cm1, x, w1l_od, b1_od, w1r_od,
             w1l_dt, b1_dt, w1r_dt, h_out, d1_out):
    cnt = jnp.maximum(cm0[:, :1] + cm1[:, :1], 1.0)
    agg = (pa0[...] + pa1[...]) / cnt
    xb = x[...]
    h_out[...] = jnp.maximum(_MM(agg, w1l_od[...]) + b1_od[...]
                             + _MM(xb, w1r_od[...]), 0.0)
    d1_out[...] = jnp.maximum(_MM(agg, w1l_dt[...]) + b1_dt[...]
                              + _MM(xb, w1r_dt[...]), 0.0)


def _t1(pa0, pa1, cm0, cm1, x, w1l_od, b1_od, w1r_od, w1l_dt, b1_dt, w1r_dt):
    grid = N // BM
    blk_d = pl.BlockSpec((BM, D), lambda i: (i, 0))
    blk_w = pl.BlockSpec((D, D), lambda i: (0, 0))
    blk_b = pl.BlockSpec((1, D), lambda i: (0, 0))
    return pl.pallas_call(
        _t1_body,
        grid=(grid,),
        in_specs=[blk_d, blk_d, blk_d, blk_d, blk_d,
                  blk_w, blk_b, blk_w, blk_w, blk_b, blk_w],
        out_specs=[blk_d, blk_d],
        out_shape=[jax.ShapeDtypeStruct((N, D), jnp.float32),
                   jax.ShapeDtypeStruct((N, D), jnp.float32)],
    )(pa0, pa1, cm0, cm1, x, w1l_od, b1_od, w1r_od, w1l_dt, b1_dt, w1r_dt)


def _t2_body(pb0, pb1, cr0, cr1, pc0, pc1, cm0, cm1, xod, d1,
             w2l, b2, w2r, w3l, b3, w3r, wlin_od, blin_od,
             dw2l, db2, dw2r, wlin_dt, blin_dt,
             w1a, w1b, db1, uod_out, udt_out):
    cntb = jnp.maximum(cr0[:, :1] + cr1[:, :1], 1.0)
    aggh = (pb0[...] + pb1[...]) / cntb
    cntc = jnp.maximum(cm0[:, :1] + cm1[:, :1], 1.0)
    aggd = (pc0[...] + pc1[...]) / cntc
    xb = xod[...]
    d1b = d1[...]
    od1 = jnp.maximum(_MM(aggh, w2l[...]) + b2[...] + _MM(xb, w2r[...]), 0.0)
    od2 = jnp.maximum(_MM(aggh, w3l[...]) + b3[...] + _MM(od1, w3r[...]), 0.0)
    d2 = jnp.maximum(_MM(aggd, dw2l[...]) + db2[...] + _MM(d1b, dw2r[...]),
                     0.0)
    z_od = _MM(od2, wlin_od[...]) + blin_od[...]
    z_dt = _MM(d2, wlin_dt[...]) + blin_dt[...]
    uod_out[...] = _MM(z_od, w1a[...])
    udt_out[...] = _MM(z_dt, w1b[...]) + db1[...]


def _t2(*args):
    grid = N // BM
    blk_d = pl.BlockSpec((BM, D), lambda i: (i, 0))
    blk_w = pl.BlockSpec((D, D), lambda i: (0, 0))
    blk_b = pl.BlockSpec((1, D), lambda i: (0, 0))
    wspecs = [blk_w, blk_b, blk_w, blk_w, blk_b, blk_w, blk_w, blk_b,
              blk_w, blk_b, blk_w, blk_w, blk_b, blk_w, blk_w, blk_b]
    return pl.pallas_call(
        _t2_body,
        grid=(grid,),
        in_specs=[blk_d] * 10 + wspecs,
        out_specs=[blk_d, blk_d],
        out_shape=[jax.ShapeDtypeStruct((N, D), jnp.float32),
                   jax.ShapeDtypeStruct((N, D), jnp.float32)],
    )(*args)


def _t3_body(ga, gb, w2row, b2s, out):
    t = jnp.maximum(ga[...] + gb[...], 0.0)
    s = jnp.sum(t * w2row[...], axis=1, keepdims=True) + b2s[...]
    out[...] = jax.nn.sigmoid(s)


def _t3(ga, gb, w2row, b2s):
    grid = LPAD // DM
    blk = pl.BlockSpec((DM, D), lambda i: (i, 0))
    return pl.pallas_call(
        _t3_body,
        grid=(grid,),
        in_specs=[blk, blk, pl.BlockSpec((1, D), lambda i: (0, 0)),
                  pl.BlockSpec((1, 1), lambda i: (0, 0))],
        out_specs=pl.BlockSpec((DM, 1), lambda i: (i, 0)),
        out_shape=jax.ShapeDtypeStruct((LPAD, 1), jnp.float32),
    )(ga, gb, w2row, b2s)


def _pad_edges(ei, pad_len, dummy_dst):
    src = jnp.concatenate([ei[0], jnp.zeros((pad_len,), jnp.int32)])
    dst = jnp.concatenate([ei[1], jnp.full((pad_len,), dummy_dst, jnp.int32)])
    return src.reshape(-1, 128), dst.reshape(-1, 128)


def kernel(x_detector_time, x_od, od_W1l, od_b1, od_W1r, od_W2l, od_b2,
           od_W2r, od_W3l, od_b3, od_W3r, od_Wlin, od_blin, dt_W1l, dt_b1,
           dt_W1r, dt_W2l, dt_b2, dt_W2r, dt_Wlin, dt_blin, dec_W1, dec_b1,
           dec_W2, dec_b2, edge_index_metapath, edge_index_rev_assignment,
           edge_label_index):
    f32 = jnp.float32
    zeros_hbm = jnp.zeros((RPS, D), f32)
    ones_hbm = jnp.ones((128, D), f32)

    src_m, dst_m = _pad_edges(edge_index_metapath, EPAD - E, N)
    src_r, dst_r = _pad_edges(edge_index_rev_assignment, EPAD - E, N)
    row_l, col_l = _pad_edges(edge_label_index, LPAD - L, 0)

    agg = _make_agg()
    cnt = _make_cnt()
    dec = _make_dec()

    row2 = lambda b: b.reshape(1, -1)

    cnts = cnt(dst_m, dst_r, zeros_hbm, ones_hbm)
    pa = agg(x_detector_time, src_m, dst_m, zeros_hbm)
    h, d1 = _t1(pa[0], pa[1], cnts[0, 0], cnts[0, 1], x_detector_time,
                od_W1l, row2(od_b1), od_W1r,
                dt_W1l, row2(dt_b1), dt_W1r)

    pb = agg(h, src_r, dst_r, zeros_hbm)
    pc = agg(d1, src_m, dst_m, zeros_hbm)

    u_od, u_dt = _t2(pb[0], pb[1], cnts[1, 0], cnts[1, 1],
                     pc[0], pc[1], cnts[0, 0], cnts[0, 1], x_od, d1,
                     od_W2l, row2(od_b2), od_W2r,
                     od_W3l, row2(od_b3), od_W3r,
                     od_Wlin, row2(od_blin),
                     dt_W2l, row2(dt_b2), dt_W2r,
                     dt_Wlin, row2(dt_blin),
                     dec_W1[:D], dec_W1[D:], row2(dec_b1))

    pad = jnp.zeros((NPAD - N, D), f32)
    ga, gb = dec(jnp.concatenate([u_od, pad]), jnp.concatenate([u_dt, pad]),
                 row_l, col_l)
    out = _t3(ga, gb, dec_W2.reshape(1, D), dec_b2.reshape(1, 1))
    return out.reshape(-1)[:L]
